# baseline (speedup 1.0000x reference)
import functools

import jax
import jax.numpy as jnp
from jax import lax
from jax.experimental import pallas as pl
from jax.experimental.pallas import tpu as pltpu
from jax.experimental.pallas import tpu_sc as plsc

N_SRC = 10000
N_DST = 10000
E = 320000
C = 128

NC = 2
NS = 16
NW = NC * NS

CHUNK = 128
NCHUNKS = E // CHUNK
BASE = NCHUNKS // NW
REM = NCHUNKS % NW
N_DST_PAD = 10112
ROWS_PER_TILE = N_DST_PAD // NS

_mesh = plsc.VectorSubcoreMesh(core_axis_name="c", subcore_axis_name="s")


def _worker_range():
    w = lax.axis_index("s") * NC + lax.axis_index("c")
    lo = w * BASE + jnp.minimum(w, REM)
    n = BASE + jnp.where(w < REM, 1, 0)
    return lo, n


@functools.partial(
    pl.kernel,
    out_type=jax.ShapeDtypeStruct((NCHUNKS, CHUNK, C), jnp.float32),
    mesh=_mesh,
    scratch_types=[
        pltpu.VMEM((2, CHUNK), jnp.int32),
        pltpu.VMEM((2, CHUNK), jnp.int32),
        pltpu.VMEM((2, CHUNK), jnp.int32),
        pltpu.VMEM((CHUNK, C), jnp.float32),
        pltpu.VMEM((CHUNK, C), jnp.float32),
        pltpu.VMEM((CHUNK, C), jnp.float32),
        pltpu.VMEM((CHUNK, C), jnp.float32),
        pltpu.VMEM((CHUNK, C), jnp.float32),
        pltpu.VMEM((CHUNK, C), jnp.float32),
        pltpu.SemaphoreType.DMA,
        pltpu.SemaphoreType.DMA,
        pltpu.SemaphoreType.DMA,
        pltpu.SemaphoreType.DMA,
        pltpu.SemaphoreType.DMA,
        pltpu.SemaphoreType.DMA,
    ],
)
def _sc_gather(pd_hbm, ps_hbm, ei_hbm, out_hbm,
               ix0, ix1, ix2, ra0, rb0, ra1, rb1, ra2, rb2,
               sem0, sem1, sem2, semw0, semw1, semw2):
    lo, n = _worker_range()
    bufs = ((ix0, ra0, rb0, sem0, semw0),
            (ix1, ra1, rb1, sem1, semw1),
            (ix2, ra2, rb2, sem2, semw2))

    def issue(i, ix, ra, rb, sem, semw):
        pltpu.sync_copy(ei_hbm.at[:, pl.ds((lo + i) * CHUNK, CHUNK)], ix)
        pltpu.async_copy(pd_hbm.at[ix.at[1]], ra, sem)
        pltpu.async_copy(ps_hbm.at[ix.at[0]], rb, sem)

    def drain_gather(b):
        ix, ra, rb, sem, semw = bufs[b]
        pltpu.make_async_copy(pd_hbm.at[ix.at[1]], ra, sem).wait()
        pltpu.make_async_copy(ps_hbm.at[ix.at[0]], rb, sem).wait()

    def drain_write(b):
        ix, ra, rb, sem, semw = bufs[b]
        pltpu.make_async_copy(rb, out_hbm.at[lo], semw).wait()

    issue(0, *bufs[0])
    issue(1, *bufs[1])

    def triple(k, carry):
        for b in (0, 1, 2):
            i = 3 * k + b

            @pl.when(i < n)
            def _():
                ix, ra, rb, sem, semw = bufs[b]
                drain_gather(b)

                def add_rows(r, c2):
                    for rr in (0, 1):
                        for cc in range(C // 16):
                            sl = pl.ds(cc * 16, 16)
                            rb[2 * r + rr, sl] = (ra[2 * r + rr, sl]
                                                  + rb[2 * r + rr, sl])
                    return c2

                lax.fori_loop(0, CHUNK // 2, add_rows, 0)
                pltpu.async_copy(rb, out_hbm.at[lo + i], semw)

                @pl.when(i + 2 < n)
                def _():
                    b2 = (b + 2) % 3
                    if b == 0:
                        @pl.when(k > 0)
                        def _():
                            drain_write(b2)
                    else:
                        drain_write(b2)
                    issue(i + 2, *bufs[b2])

        return carry

    lax.fori_loop(0, (BASE + 4) // 3, triple, 0)
    drain_write(0)
    drain_write(1)
    drain_write(2)


@functools.partial(
    pl.kernel,
    out_type=jax.ShapeDtypeStruct((NC, N_DST_PAD, C), jnp.float32),
    mesh=_mesh,
    scratch_types=[
        pltpu.VMEM((CHUNK,), jnp.int32),
        pltpu.VMEM((CHUNK,), jnp.int32),
        pltpu.VMEM((CHUNK,), jnp.int32),
        pltpu.VMEM((CHUNK, C), jnp.float32),
        pltpu.VMEM((CHUNK, C), jnp.float32),
        pltpu.VMEM((CHUNK, C), jnp.float32),
        pltpu.VMEM_SHARED((N_DST_PAD, C), jnp.float32),
        pltpu.SemaphoreType.DMA,
        pltpu.SemaphoreType.DMA,
        pltpu.SemaphoreType.DMA,
        pltpu.SemaphoreType.DMA,
        pltpu.SemaphoreType.DMA,
        pltpu.SemaphoreType.DMA,
    ],
)
def _sc_scatter(en_hbm, dsti_hbm, zeros_hbm, out_hbm,
                di0, di1, di2, rows0, rows1, rows2, acc_sh,
                semi0, semi1, semi2, semsc0, semsc1, semsc2):
    c = lax.axis_index("c")
    s = lax.axis_index("s")
    w = s * NC + c
    lo = w * BASE + jnp.minimum(w, REM)
    n = BASE + jnp.where(w < REM, 1, 0)
    bufs = ((di0, rows0, semi0, semsc0),
            (di1, rows1, semi1, semsc1),
            (di2, rows2, semi2, semsc2))

    row_sl = pl.ds(s * ROWS_PER_TILE, ROWS_PER_TILE)
    pltpu.sync_copy(zeros_hbm.at[row_sl], acc_sh.at[row_sl])
    plsc.subcore_barrier()

    def issue(i, di, rows, semi, semsc):
        pltpu.sync_copy(dsti_hbm.at[pl.ds((lo + i) * CHUNK, CHUNK)], di)
        pltpu.async_copy(en_hbm.at[lo + i], rows, semi)

    def drain_sc(b):
        di, rows, semi, semsc = bufs[b]
        pltpu.make_async_copy(rows, acc_sh.at[di], semsc).wait()

    issue(0, *bufs[0])
    issue(1, *bufs[1])

    def triple(k, carry):
        for b in (0, 1, 2):
            i = 3 * k + b

            @pl.when(i < n)
            def _():
                di, rows, semi, semsc = bufs[b]
                pltpu.make_async_copy(en_hbm.at[lo + i], rows, semi).wait()
                pltpu.async_copy(rows, acc_sh.at[di], semsc, add=True)

                @pl.when(i + 2 < n)
                def _():
                    b2 = (b + 2) % 3
                    if b == 0:
                        @pl.when(k > 0)
                        def _():
                            drain_sc(b2)
                    else:
                        drain_sc(b2)
                    issue(i + 2, *bufs[b2])

        return carry

    lax.fori_loop(0, (BASE + 4) // 3, triple, 0)
    drain_sc(0)
    drain_sc(1)
    drain_sc(2)
    plsc.subcore_barrier()
    pltpu.sync_copy(acc_sh.at[row_sl], out_hbm.at[c, row_sl])


def _layer_norm(h, scale, bias, eps=1e-5):
    mu = jnp.mean(h, axis=-1, keepdims=True)
    xc = h - mu
    var = jnp.mean(xc * xc, axis=-1, keepdims=True)
    return xc * lax.rsqrt(var + eps) * scale + bias


def _tc_pre_body(xs_ref, xd_ref, w1i_ref, w1j_ref, b1_ref,
                 nw1_ref, nb1_ref, nw2_ref, nb2_ref, lns_ref, lnb_ref,
                 pd_ref, ps_ref, ns_ref):
    xs = xs_ref[...]
    xd = xd_ref[...]
    f32 = jnp.float32
    pd_ref[...] = jnp.dot(xd, w1i_ref[...], preferred_element_type=f32) + b1_ref[...]
    ps_ref[...] = jnp.dot(xs, w1j_ref[...], preferred_element_type=f32)
    nw1s = nw1_ref[0:C, :] + nw1_ref[C:2 * C, :]
    h = jax.nn.silu(jnp.dot(xs, nw1s, preferred_element_type=f32) + nb1_ref[...])
    h2 = jnp.dot(h, nw2_ref[...], preferred_element_type=f32) + nb2_ref[...]
    ns_ref[...] = _layer_norm(h2, lns_ref[...], lnb_ref[...]) + xs


def _tc_edge_body(hp_ref, ea_ref, w1e_ref, w2_ref, b2_ref,
                  lns_ref, lnb_ref, out_ref):
    f32 = jnp.float32
    ea = ea_ref[...]
    h1 = hp_ref[...] + jnp.dot(ea, w1e_ref[...], preferred_element_type=f32)
    h = jax.nn.silu(h1)
    e2 = jnp.dot(h, w2_ref[...], preferred_element_type=f32) + b2_ref[...]
    out_ref[...] = _layer_norm(e2, lns_ref[...], lnb_ref[...]) + ea


def _tc_post_body(xd_ref, p0_ref, p1_ref, nw1_ref, nb1_ref, nw2_ref, nb2_ref,
                  lns_ref, lnb_ref, nd_ref):
    f32 = jnp.float32
    xd = xd_ref[...]
    agg = p0_ref[...] + p1_ref[...]
    h = jnp.dot(xd, nw1_ref[0:C, :], preferred_element_type=f32)
    h = h + jnp.dot(agg, nw1_ref[C:2 * C, :], preferred_element_type=f32)
    h = jax.nn.silu(h + nb1_ref[...])
    h2 = jnp.dot(h, nw2_ref[...], preferred_element_type=f32) + nb2_ref[...]
    nd_ref[...] = _layer_norm(h2, lns_ref[...], lnb_ref[...]) + xd


EBLK = 16000


def kernel(x_src, x_dst, edge_attr, edge_index,
           edge_W1, edge_b1, edge_W2, edge_b2, edge_ln_s, edge_ln_b,
           node_W1, node_b1, node_W2, node_b2, node_ln_s, node_ln_b):
    f32 = jnp.float32
    dst = edge_index[1]
    w1i = edge_W1[0:C, :]
    w1j = edge_W1[C:2 * C, :]
    w1e = edge_W1[2 * C:3 * C, :]
    b1 = edge_b1.reshape(1, C)
    b2 = edge_b2.reshape(1, C)
    elns = edge_ln_s.reshape(1, C)
    elnb = edge_ln_b.reshape(1, C)
    nb1 = node_b1.reshape(1, C)
    nb2 = node_b2.reshape(1, C)
    nlns = node_ln_s.reshape(1, C)
    nlnb = node_ln_b.reshape(1, C)

    full = lambda shape: pl.BlockSpec(shape, lambda: (0,) * len(shape))
    p_dst, p_src, nodes_new_src = pl.pallas_call(
        _tc_pre_body,
        out_shape=[
            jax.ShapeDtypeStruct((N_DST, C), f32),
            jax.ShapeDtypeStruct((N_SRC, C), f32),
            jax.ShapeDtypeStruct((N_SRC, C), f32),
        ],
        in_specs=[full((N_SRC, C)), full((N_DST, C)),
                  full((C, C)), full((C, C)), full((1, C)),
                  full((2 * C, C)), full((1, C)), full((C, C)), full((1, C)),
                  full((1, C)), full((1, C))],
        out_specs=[full((N_DST, C)), full((N_SRC, C)), full((N_SRC, C))],
    )(x_src, x_dst, w1i, w1j, b1,
      node_W1, nb1, node_W2, nb2, nlns, nlnb)

    h_pre = _sc_gather(p_dst, p_src, edge_index)

    h_pre2 = h_pre.reshape(E, C)
    eblk = lambda: pl.BlockSpec((EBLK, C), lambda i: (i, 0))
    wfull = lambda shape: pl.BlockSpec(shape, lambda i: (0,) * len(shape))
    edges_new = pl.pallas_call(
        _tc_edge_body,
        grid=(E // EBLK,),
        out_shape=jax.ShapeDtypeStruct((E, C), f32),
        in_specs=[eblk(), eblk(), wfull((C, C)), wfull((C, C)),
                  wfull((1, C)), wfull((1, C)), wfull((1, C))],
        out_specs=eblk(),
    )(h_pre2, edge_attr, w1e, edge_W2, b2, elns, elnb)

    zeros = jnp.zeros((N_DST_PAD, C), f32)
    partials = _sc_scatter(edges_new.reshape(NCHUNKS, CHUNK, C), dst, zeros)

    nodes_new_dst = pl.pallas_call(
        _tc_post_body,
        out_shape=jax.ShapeDtypeStruct((N_DST, C), f32),
        in_specs=[full((N_DST, C)), full((N_DST, C)), full((N_DST, C)),
                  full((2 * C, C)), full((1, C)), full((C, C)), full((1, C)),
                  full((1, C)), full((1, C))],
        out_specs=full((N_DST, C)),
    )(x_dst, partials[0, :N_DST], partials[1, :N_DST],
      node_W1, nb1, node_W2, nb2, nlns, nlnb)

    return (nodes_new_src, nodes_new_dst, edges_new)


# speedup vs baseline: 1.0042x; 1.0042x over previous
import functools

import jax
import jax.numpy as jnp
from jax import lax
from jax.experimental import pallas as pl
from jax.experimental.pallas import tpu as pltpu
from jax.experimental.pallas import tpu_sc as plsc

N_SRC = 10000
N_DST = 10000
E = 320000
C = 128

NC = 2
NS = 16
NW = NC * NS

CHUNK = 128
NCHUNKS = E // CHUNK
BASE = NCHUNKS // NW
REM = NCHUNKS % NW
N_DST_PAD = 10112
ROWS_PER_TILE = N_DST_PAD // NS

_mesh = plsc.VectorSubcoreMesh(core_axis_name="c", subcore_axis_name="s")


def _worker_range():
    w = lax.axis_index("s") * NC + lax.axis_index("c")
    lo = w * BASE + jnp.minimum(w, REM)
    n = BASE + jnp.where(w < REM, 1, 0)
    return lo, n


@functools.partial(
    pl.kernel,
    out_type=jax.ShapeDtypeStruct((NCHUNKS, CHUNK, C), jnp.float32),
    mesh=_mesh,
    scratch_types=[
        pltpu.VMEM((2, CHUNK), jnp.int32),
        pltpu.VMEM((2, CHUNK), jnp.int32),
        pltpu.VMEM((2, CHUNK), jnp.int32),
        pltpu.VMEM((CHUNK, C), jnp.float32),
        pltpu.VMEM((CHUNK, C), jnp.float32),
        pltpu.VMEM((CHUNK, C), jnp.float32),
        pltpu.VMEM((CHUNK, C), jnp.float32),
        pltpu.VMEM((CHUNK, C), jnp.float32),
        pltpu.VMEM((CHUNK, C), jnp.float32),
        pltpu.SemaphoreType.DMA,
        pltpu.SemaphoreType.DMA,
        pltpu.SemaphoreType.DMA,
        pltpu.SemaphoreType.DMA,
        pltpu.SemaphoreType.DMA,
        pltpu.SemaphoreType.DMA,
    ],
)
def _sc_gather(pd_hbm, ps_hbm, ei_hbm, out_hbm,
               ix0, ix1, ix2, ra0, rb0, ra1, rb1, ra2, rb2,
               sem0, sem1, sem2, semw0, semw1, semw2):
    lo, n = _worker_range()
    bufs = ((ix0, ra0, rb0, sem0, semw0),
            (ix1, ra1, rb1, sem1, semw1),
            (ix2, ra2, rb2, sem2, semw2))

    def issue(i, ix, ra, rb, sem, semw):
        pltpu.sync_copy(ei_hbm.at[:, pl.ds((lo + i) * CHUNK, CHUNK)], ix)
        pltpu.async_copy(pd_hbm.at[ix.at[1]], ra, sem)
        pltpu.async_copy(ps_hbm.at[ix.at[0]], rb, sem)

    def drain_gather(b):
        ix, ra, rb, sem, semw = bufs[b]
        pltpu.make_async_copy(pd_hbm.at[ix.at[1]], ra, sem).wait()
        pltpu.make_async_copy(ps_hbm.at[ix.at[0]], rb, sem).wait()

    def drain_write(b):
        ix, ra, rb, sem, semw = bufs[b]
        pltpu.make_async_copy(rb, out_hbm.at[lo], semw).wait()

    issue(0, *bufs[0])
    issue(1, *bufs[1])

    def triple(k, carry):
        for b in (0, 1, 2):
            i = 3 * k + b

            @pl.when(i < n)
            def _():
                ix, ra, rb, sem, semw = bufs[b]
                drain_gather(b)

                def add_rows(r, c2):
                    for rr in (0, 1):
                        for cc in range(C // 16):
                            sl = pl.ds(cc * 16, 16)
                            rb[2 * r + rr, sl] = (ra[2 * r + rr, sl]
                                                  + rb[2 * r + rr, sl])
                    return c2

                lax.fori_loop(0, CHUNK // 2, add_rows, 0)
                pltpu.async_copy(rb, out_hbm.at[lo + i], semw)

                @pl.when(i + 2 < n)
                def _():
                    b2 = (b + 2) % 3
                    if b == 0:
                        @pl.when(k > 0)
                        def _():
                            drain_write(b2)
                    else:
                        drain_write(b2)
                    issue(i + 2, *bufs[b2])

        return carry

    lax.fori_loop(0, (BASE + 4) // 3, triple, 0)
    drain_write(0)
    drain_write(1)
    drain_write(2)


@functools.partial(
    pl.kernel,
    out_type=jax.ShapeDtypeStruct((NC, N_DST_PAD, C), jnp.float32),
    mesh=_mesh,
    scratch_types=[
        pltpu.VMEM((CHUNK,), jnp.int32),
        pltpu.VMEM((CHUNK,), jnp.int32),
        pltpu.VMEM((CHUNK,), jnp.int32),
        pltpu.VMEM((CHUNK, C), jnp.float32),
        pltpu.VMEM((CHUNK, C), jnp.float32),
        pltpu.VMEM((CHUNK, C), jnp.float32),
        pltpu.VMEM_SHARED((N_DST_PAD, C), jnp.float32),
        pltpu.SemaphoreType.DMA,
        pltpu.SemaphoreType.DMA,
        pltpu.SemaphoreType.DMA,
        pltpu.SemaphoreType.DMA,
        pltpu.SemaphoreType.DMA,
        pltpu.SemaphoreType.DMA,
    ],
)
def _sc_scatter(en_hbm, dsti_hbm, zeros_hbm, out_hbm,
                di0, di1, di2, rows0, rows1, rows2, acc_sh,
                semi0, semi1, semi2, semsc0, semsc1, semsc2):
    c = lax.axis_index("c")
    s = lax.axis_index("s")
    w = s * NC + c
    lo = w * BASE + jnp.minimum(w, REM)
    n = BASE + jnp.where(w < REM, 1, 0)
    bufs = ((di0, rows0, semi0, semsc0),
            (di1, rows1, semi1, semsc1),
            (di2, rows2, semi2, semsc2))

    row_sl = pl.ds(s * ROWS_PER_TILE, ROWS_PER_TILE)
    pltpu.sync_copy(zeros_hbm.at[row_sl], acc_sh.at[row_sl])
    plsc.subcore_barrier()

    def issue(i, di, rows, semi, semsc):
        pltpu.sync_copy(dsti_hbm.at[pl.ds((lo + i) * CHUNK, CHUNK)], di)
        pltpu.async_copy(en_hbm.at[lo + i], rows, semi)

    def drain_sc(b):
        di, rows, semi, semsc = bufs[b]
        pltpu.make_async_copy(rows, acc_sh.at[di], semsc).wait()

    issue(0, *bufs[0])
    issue(1, *bufs[1])

    def triple(k, carry):
        for b in (0, 1, 2):
            i = 3 * k + b

            @pl.when(i < n)
            def _():
                di, rows, semi, semsc = bufs[b]
                pltpu.make_async_copy(en_hbm.at[lo + i], rows, semi).wait()
                pltpu.async_copy(rows, acc_sh.at[di], semsc, add=True)

                @pl.when(i + 2 < n)
                def _():
                    b2 = (b + 2) % 3
                    if b == 0:
                        @pl.when(k > 0)
                        def _():
                            drain_sc(b2)
                    else:
                        drain_sc(b2)
                    issue(i + 2, *bufs[b2])

        return carry

    lax.fori_loop(0, (BASE + 4) // 3, triple, 0)
    drain_sc(0)
    drain_sc(1)
    drain_sc(2)
    plsc.subcore_barrier()
    pltpu.sync_copy(acc_sh.at[row_sl], out_hbm.at[c, row_sl])


def _layer_norm(h, scale, bias, eps=1e-5):
    mu = jnp.mean(h, axis=-1, keepdims=True)
    xc = h - mu
    var = jnp.mean(xc * xc, axis=-1, keepdims=True)
    return xc * lax.rsqrt(var + eps) * scale + bias


def _tc_pre_body(xs_ref, xd_ref, w1i_ref, w1j_ref, b1_ref,
                 nw1_ref, nb1_ref, nw2_ref, nb2_ref, lns_ref, lnb_ref,
                 pd_ref, ps_ref, ns_ref):
    xs = xs_ref[...]
    xd = xd_ref[...]
    f32 = jnp.float32
    pd_ref[...] = jnp.dot(xd, w1i_ref[...], preferred_element_type=f32) + b1_ref[...]
    ps_ref[...] = jnp.dot(xs, w1j_ref[...], preferred_element_type=f32)
    nw1s = nw1_ref[0:C, :] + nw1_ref[C:2 * C, :]
    h = jax.nn.silu(jnp.dot(xs, nw1s, preferred_element_type=f32) + nb1_ref[...])
    h2 = jnp.dot(h, nw2_ref[...], preferred_element_type=f32) + nb2_ref[...]
    ns_ref[...] = _layer_norm(h2, lns_ref[...], lnb_ref[...]) + xs


def _tc_edge_body(hp_ref, ea_ref, w1e_ref, w2_ref, b2_ref,
                  lns_ref, lnb_ref, out_ref):
    f32 = jnp.float32
    ea = ea_ref[...]
    h1 = hp_ref[...] + jnp.dot(ea, w1e_ref[...], preferred_element_type=f32)
    h = jax.nn.silu(h1)
    e2 = jnp.dot(h, w2_ref[...], preferred_element_type=f32) + b2_ref[...]
    out_ref[...] = _layer_norm(e2, lns_ref[...], lnb_ref[...]) + ea


def _tc_post_body(xd_ref, p0_ref, p1_ref, nw1_ref, nb1_ref, nw2_ref, nb2_ref,
                  lns_ref, lnb_ref, nd_ref):
    f32 = jnp.float32
    xd = xd_ref[...]
    agg = p0_ref[...] + p1_ref[...]
    h = jnp.dot(xd, nw1_ref[0:C, :], preferred_element_type=f32)
    h = h + jnp.dot(agg, nw1_ref[C:2 * C, :], preferred_element_type=f32)
    h = jax.nn.silu(h + nb1_ref[...])
    h2 = jnp.dot(h, nw2_ref[...], preferred_element_type=f32) + nb2_ref[...]
    nd_ref[...] = _layer_norm(h2, lns_ref[...], lnb_ref[...]) + xd


EBLK = 8000


def kernel(x_src, x_dst, edge_attr, edge_index,
           edge_W1, edge_b1, edge_W2, edge_b2, edge_ln_s, edge_ln_b,
           node_W1, node_b1, node_W2, node_b2, node_ln_s, node_ln_b):
    f32 = jnp.float32
    dst = edge_index[1]
    w1i = edge_W1[0:C, :]
    w1j = edge_W1[C:2 * C, :]
    w1e = edge_W1[2 * C:3 * C, :]
    b1 = edge_b1.reshape(1, C)
    b2 = edge_b2.reshape(1, C)
    elns = edge_ln_s.reshape(1, C)
    elnb = edge_ln_b.reshape(1, C)
    nb1 = node_b1.reshape(1, C)
    nb2 = node_b2.reshape(1, C)
    nlns = node_ln_s.reshape(1, C)
    nlnb = node_ln_b.reshape(1, C)

    full = lambda shape: pl.BlockSpec(shape, lambda: (0,) * len(shape))
    p_dst, p_src, nodes_new_src = pl.pallas_call(
        _tc_pre_body,
        out_shape=[
            jax.ShapeDtypeStruct((N_DST, C), f32),
            jax.ShapeDtypeStruct((N_SRC, C), f32),
            jax.ShapeDtypeStruct((N_SRC, C), f32),
        ],
        in_specs=[full((N_SRC, C)), full((N_DST, C)),
                  full((C, C)), full((C, C)), full((1, C)),
                  full((2 * C, C)), full((1, C)), full((C, C)), full((1, C)),
                  full((1, C)), full((1, C))],
        out_specs=[full((N_DST, C)), full((N_SRC, C)), full((N_SRC, C))],
    )(x_src, x_dst, w1i, w1j, b1,
      node_W1, nb1, node_W2, nb2, nlns, nlnb)

    h_pre = _sc_gather(p_dst, p_src, edge_index)

    h_pre2 = h_pre.reshape(E, C)
    eblk = lambda: pl.BlockSpec((EBLK, C), lambda i: (i, 0))
    wfull = lambda shape: pl.BlockSpec(shape, lambda i: (0,) * len(shape))
    edges_new = pl.pallas_call(
        _tc_edge_body,
        grid=(E // EBLK,),
        out_shape=jax.ShapeDtypeStruct((E, C), f32),
        in_specs=[eblk(), eblk(), wfull((C, C)), wfull((C, C)),
                  wfull((1, C)), wfull((1, C)), wfull((1, C))],
        out_specs=eblk(),
    )(h_pre2, edge_attr, w1e, edge_W2, b2, elns, elnb)

    zeros = jnp.zeros((N_DST_PAD, C), f32)
    partials = _sc_scatter(edges_new.reshape(NCHUNKS, CHUNK, C), dst, zeros)

    nodes_new_dst = pl.pallas_call(
        _tc_post_body,
        out_shape=jax.ShapeDtypeStruct((N_DST, C), f32),
        in_specs=[full((N_DST, C)), full((N_DST, C)), full((N_DST, C)),
                  full((2 * C, C)), full((1, C)), full((C, C)), full((1, C)),
                  full((1, C)), full((1, C))],
        out_specs=full((N_DST, C)),
    )(x_dst, partials[0, :N_DST], partials[1, :N_DST],
      node_W1, nb1, node_W2, nb2, nlns, nlnb)

    return (nodes_new_src, nodes_new_dst, edges_new)
